# baseline (speedup 1.0000x reference)
import functools

import jax
import jax.numpy as jnp
from jax import lax
from jax.experimental import pallas as pl
from jax.experimental.pallas import tpu as pltpu
from jax.experimental.pallas import tpu_sc as plsc

EMBED_DIM = 64
_NUM_CORES = 2
_NUM_SUBCORES = 16
_NW = _NUM_CORES * _NUM_SUBCORES
_CHUNK = 256
_LANES = 16


def _make_gather(hist, batch):
    stripe = batch // _NW
    per_h = stripe // _CHUNK
    nstep = hist * per_h
    assert nstep % 2 == 0 and nstep >= 4
    mesh = plsc.VectorSubcoreMesh(core_axis_name="c", subcore_axis_name="s")

    @functools.partial(
        pl.kernel,
        mesh=mesh,
        out_type=jax.ShapeDtypeStruct((hist, EMBED_DIM, batch), jnp.float32),
        scratch_types=[
            pltpu.VMEM((_CHUNK,), jnp.int32),
            pltpu.VMEM((_CHUNK,), jnp.int32),
            pltpu.VMEM((_CHUNK, EMBED_DIM), jnp.float32),
            pltpu.VMEM((_CHUNK, EMBED_DIM), jnp.float32),
            pltpu.VMEM((EMBED_DIM, _CHUNK), jnp.float32),
            pltpu.VMEM((EMBED_DIM, _CHUNK), jnp.float32),
            pltpu.SemaphoreType.DMA,
            pltpu.SemaphoreType.DMA,
            pltpu.SemaphoreType.DMA,
            pltpu.SemaphoreType.DMA,
            pltpu.SemaphoreType.DMA,
            pltpu.SemaphoreType.DMA,
        ],
        compiler_params=pltpu.CompilerParams(
            use_tc_tiling_on_sc=False, needs_layout_passes=False
        ),
    )
    def gather_kernel(idx_hbm, table_hbm, out_hbm,
                      idx0, idx1, rows0, rows1, tb0, tb1,
                      si0, si1, sg0, sg1, sw0, sw1):
        wid = lax.axis_index("s") * _NUM_CORES + lax.axis_index("c")
        b_base = wid * stripe
        idx_b, rows_b, tb_b = (idx0, idx1), (rows0, rows1), (tb0, tb1)
        si, sg, sw = (si0, si1), (sg0, sg1), (sw0, sw1)

        def chunk_pos(m):
            return m // per_h, b_base + (m % per_h) * _CHUNK

        def start_idx(m, b):
            h, b0 = chunk_pos(m)
            pltpu.make_async_copy(
                idx_hbm.at[h, pl.ds(b0, _CHUNK)], idx_b[b], si[b]
            ).start()

        def wait_idx(b):
            pltpu.make_async_copy(
                idx_hbm.at[0, pl.ds(0, _CHUNK)], idx_b[b], si[b]
            ).wait()

        def start_gather(b):
            pltpu.make_async_copy(table_hbm.at[idx_b[b]], rows_b[b], sg[b]).start()

        def wait_gather(b):
            pltpu.make_async_copy(table_hbm.at[idx_b[b]], rows_b[b], sg[b]).wait()

        def transpose(b):
            rows, tb = rows_b[b], tb_b[b]
            lane = lax.iota(jnp.int32, _LANES)

            def body(i, carry):
                r0 = i * _LANES
                for c in range(EMBED_DIM):
                    v = plsc.load_gather(
                        rows, [r0 + lane, jnp.full((_LANES,), c, jnp.int32)]
                    )
                    tb[c, pl.ds(r0, _LANES)] = v
                return carry

            lax.fori_loop(0, _CHUNK // _LANES, body, 0)

        def start_wb(m, b):
            h, b0 = chunk_pos(m)
            pltpu.make_async_copy(
                tb_b[b], out_hbm.at[h].at[:, pl.ds(b0, _CHUNK)], sw[b]
            ).start()

        def wait_wb(b):
            pltpu.make_async_copy(
                tb_b[b], out_hbm.at[0].at[:, pl.ds(0, _CHUNK)], sw[b]
            ).wait()

        def steady(m, b, prefetch=True):
            o = 1 - b
            wait_idx(b)
            wait_wb(b)
            start_gather(b)
            wait_gather(o)
            if prefetch:
                start_idx(m + 1, o)
            transpose(o)
            start_wb(m - 1, o)

        start_idx(0, 0)
        wait_idx(0)
        start_gather(0)
        start_idx(1, 1)
        wait_idx(1)
        start_gather(1)
        wait_gather(0)
        start_idx(2, 0)
        transpose(0)
        start_wb(0, 0)

        def pair(p, carry):
            m = 2 * p
            steady(m, 0)
            steady(m + 1, 1)
            return carry

        lax.fori_loop(1, nstep // 2 - 1, pair, 0)

        steady(nstep - 2, 0)
        steady(nstep - 1, 1, prefetch=False)
        wait_gather(1)
        transpose(1)
        start_wb(nstep - 1, 1)
        wait_wb(0)
        wait_wb(1)

    return gather_kernel


def kernel(indices, table):
    batch, hist = indices.shape
    idx_t = indices.T.astype(jnp.int32)
    out_t = _make_gather(hist, batch)(idx_t, table)
    return jnp.transpose(out_t, (2, 0, 1))


# speedup vs baseline: 1.1253x; 1.1253x over previous
import functools

import jax
import jax.numpy as jnp
from jax import lax
from jax.experimental import pallas as pl
from jax.experimental.pallas import tpu as pltpu
from jax.experimental.pallas import tpu_sc as plsc

EMBED_DIM = 64
_NUM_CORES = 2
_NUM_SUBCORES = 16
_NW = _NUM_CORES * _NUM_SUBCORES
_CHUNK = 256
_LANES = 16


def _make_gather(hist, batch):
    stripe = batch // _NW
    per_h = stripe // _CHUNK
    nstep = hist * per_h
    assert nstep % 2 == 0 and nstep >= 4
    mesh = plsc.VectorSubcoreMesh(core_axis_name="c", subcore_axis_name="s")

    @functools.partial(
        pl.kernel,
        mesh=mesh,
        out_type=jax.ShapeDtypeStruct((hist, EMBED_DIM, batch), jnp.float32),
        scratch_types=[
            pltpu.VMEM((_CHUNK,), jnp.int32),
            pltpu.VMEM((_CHUNK,), jnp.int32),
            pltpu.VMEM((_CHUNK, EMBED_DIM), jnp.float32),
            pltpu.VMEM((_CHUNK, EMBED_DIM), jnp.float32),
            pltpu.VMEM((EMBED_DIM, _CHUNK), jnp.float32),
            pltpu.VMEM((EMBED_DIM, _CHUNK), jnp.float32),
            pltpu.SemaphoreType.DMA,
            pltpu.SemaphoreType.DMA,
            pltpu.SemaphoreType.DMA,
            pltpu.SemaphoreType.DMA,
            pltpu.SemaphoreType.DMA,
            pltpu.SemaphoreType.DMA,
        ],
        compiler_params=pltpu.CompilerParams(
            use_tc_tiling_on_sc=False, needs_layout_passes=False
        ),
    )
    def gather_kernel(idx_hbm, table_hbm, out_hbm,
                      idx0, idx1, rows0, rows1, tb0, tb1,
                      si0, si1, sg0, sg1, sw0, sw1):
        wid = lax.axis_index("s") * _NUM_CORES + lax.axis_index("c")
        b_base = wid * stripe
        idx_b, rows_b, tb_b = (idx0, idx1), (rows0, rows1), (tb0, tb1)
        si, sg, sw = (si0, si1), (sg0, sg1), (sw0, sw1)

        def chunk_pos(m):
            return m // per_h, b_base + (m % per_h) * _CHUNK

        def start_idx(m, b):
            h, b0 = chunk_pos(m)
            pltpu.make_async_copy(
                idx_hbm.at[h, pl.ds(b0, _CHUNK)], idx_b[b], si[b]
            ).start()

        def wait_idx(b):
            pltpu.make_async_copy(
                idx_hbm.at[0, pl.ds(0, _CHUNK)], idx_b[b], si[b]
            ).wait()

        def start_gather(b):
            pltpu.make_async_copy(table_hbm.at[idx_b[b]], rows_b[b], sg[b]).start()

        def wait_gather(b):
            pltpu.make_async_copy(table_hbm.at[idx_b[b]], rows_b[b], sg[b]).wait()

        def transpose(b):
            rows, tb = rows_b[b], tb_b[b]
            lane = lax.iota(jnp.int32, _LANES)

            def body(r, carry):
                rvec = jnp.full((_LANES,), r, jnp.int32)
                for cg in range(EMBED_DIM // _LANES):
                    v = rows[r, pl.ds(cg * _LANES, _LANES)]
                    plsc.store_scatter(tb, [lane + cg * _LANES, rvec], v)
                return carry

            lax.fori_loop(0, _CHUNK, body, 0)

        def start_wb(m, b):
            h, b0 = chunk_pos(m)
            pltpu.make_async_copy(
                tb_b[b], out_hbm.at[h].at[:, pl.ds(b0, _CHUNK)], sw[b]
            ).start()

        def wait_wb(b):
            pltpu.make_async_copy(
                tb_b[b], out_hbm.at[0].at[:, pl.ds(0, _CHUNK)], sw[b]
            ).wait()

        def steady(m, b, prefetch=True):
            o = 1 - b
            wait_idx(b)
            wait_wb(b)
            start_gather(b)
            wait_gather(o)
            if prefetch:
                start_idx(m + 1, o)
            transpose(o)
            start_wb(m - 1, o)

        start_idx(0, 0)
        wait_idx(0)
        start_gather(0)
        start_idx(1, 1)
        wait_idx(1)
        start_gather(1)
        wait_gather(0)
        start_idx(2, 0)
        transpose(0)
        start_wb(0, 0)

        def pair(p, carry):
            m = 2 * p
            steady(m, 0)
            steady(m + 1, 1)
            return carry

        lax.fori_loop(1, nstep // 2 - 1, pair, 0)

        steady(nstep - 2, 0)
        steady(nstep - 1, 1, prefetch=False)
        wait_gather(1)
        transpose(1)
        start_wb(nstep - 1, 1)
        wait_wb(0)
        wait_wb(1)

    return gather_kernel


def kernel(indices, table):
    batch, hist = indices.shape
    idx_t = indices.T.astype(jnp.int32)
    out_t = _make_gather(hist, batch)(idx_t, table)
    return jnp.transpose(out_t, (2, 0, 1))


# speedup vs baseline: 1.3302x; 1.1821x over previous
import functools

import jax
import jax.numpy as jnp
from jax import lax
from jax.experimental import pallas as pl
from jax.experimental.pallas import tpu as pltpu
from jax.experimental.pallas import tpu_sc as plsc

EMBED_DIM = 64
_NUM_CORES = 2
_NUM_SUBCORES = 16
_NW = _NUM_CORES * _NUM_SUBCORES
_CHUNK = 256
_LANES = 16


def _make_gather(hist, batch):
    stripe = batch // _NW
    per_h = stripe // _CHUNK
    nstep = hist * per_h
    assert nstep % 2 == 0 and nstep >= 4
    mesh = plsc.VectorSubcoreMesh(core_axis_name="c", subcore_axis_name="s")

    @functools.partial(
        pl.kernel,
        mesh=mesh,
        out_type=jax.ShapeDtypeStruct((hist, EMBED_DIM, batch), jnp.float32),
        scratch_types=[
            pltpu.VMEM((_CHUNK,), jnp.int32),
            pltpu.VMEM((_CHUNK,), jnp.int32),
            pltpu.VMEM((_CHUNK, EMBED_DIM), jnp.float32),
            pltpu.VMEM((_CHUNK, EMBED_DIM), jnp.float32),
            pltpu.VMEM((EMBED_DIM, _CHUNK), jnp.float32),
            pltpu.VMEM((EMBED_DIM, _CHUNK), jnp.float32),
            pltpu.SemaphoreType.DMA,
            pltpu.SemaphoreType.DMA,
            pltpu.SemaphoreType.DMA,
            pltpu.SemaphoreType.DMA,
            pltpu.SemaphoreType.DMA,
            pltpu.SemaphoreType.DMA,
        ],
        compiler_params=pltpu.CompilerParams(
            use_tc_tiling_on_sc=False, needs_layout_passes=False
        ),
    )
    def gather_kernel(idx_hbm, table_hbm, out_hbm,
                      idx0, idx1, rows0, rows1, tb0, tb1,
                      si0, si1, sg0, sg1, sw0, sw1):
        wid = lax.axis_index("s") * _NUM_CORES + lax.axis_index("c")
        b_base = wid * stripe
        idx_b, rows_b, tb_b = (idx0, idx1), (rows0, rows1), (tb0, tb1)
        si, sg, sw = (si0, si1), (sg0, sg1), (sw0, sw1)

        def chunk_pos(m):
            return m // per_h, b_base + (m % per_h) * _CHUNK

        def start_idx(m, b):
            h, b0 = chunk_pos(m)
            pltpu.make_async_copy(
                idx_hbm.at[h, pl.ds(b0, _CHUNK)], idx_b[b], si[b]
            ).start()

        def wait_idx(b):
            pltpu.make_async_copy(
                idx_hbm.at[0, pl.ds(0, _CHUNK)], idx_b[b], si[b]
            ).wait()

        def start_gather(b):
            pltpu.make_async_copy(table_hbm.at[idx_b[b]], rows_b[b], sg[b]).start()

        def wait_gather(b):
            pltpu.make_async_copy(table_hbm.at[idx_b[b]], rows_b[b], sg[b]).wait()

        def transpose(b):
            rows, tb = rows_b[b], tb_b[b]
            lane = lax.iota(jnp.int32, _LANES)
            rowidx = [lane + cg * _LANES for cg in range(EMBED_DIM // _LANES)]

            @plsc.parallel_loop(0, _CHUNK, unroll=8)
            def body(r):
                rvec = jnp.full((_LANES,), r, jnp.int32)
                for cg in range(EMBED_DIM // _LANES):
                    v = rows[r, pl.ds(cg * _LANES, _LANES)]
                    plsc.store_scatter(tb, [rowidx[cg], rvec], v)

        def start_wb(m, b):
            h, b0 = chunk_pos(m)
            pltpu.make_async_copy(
                tb_b[b], out_hbm.at[h].at[:, pl.ds(b0, _CHUNK)], sw[b]
            ).start()

        def wait_wb(b):
            pltpu.make_async_copy(
                tb_b[b], out_hbm.at[0].at[:, pl.ds(0, _CHUNK)], sw[b]
            ).wait()

        def steady(m, b, prefetch=True):
            o = 1 - b
            wait_idx(b)
            wait_wb(b)
            start_gather(b)
            wait_gather(o)
            if prefetch:
                start_idx(m + 1, o)
            transpose(o)
            start_wb(m - 1, o)

        start_idx(0, 0)
        wait_idx(0)
        start_gather(0)
        start_idx(1, 1)
        wait_idx(1)
        start_gather(1)
        wait_gather(0)
        start_idx(2, 0)
        transpose(0)
        start_wb(0, 0)

        def pair(p, carry):
            m = 2 * p
            steady(m, 0)
            steady(m + 1, 1)
            return carry

        lax.fori_loop(1, nstep // 2 - 1, pair, 0)

        steady(nstep - 2, 0)
        steady(nstep - 1, 1, prefetch=False)
        wait_gather(1)
        transpose(1)
        start_wb(nstep - 1, 1)
        wait_wb(0)
        wait_wb(1)

    return gather_kernel


def kernel(indices, table):
    batch, hist = indices.shape
    idx_t = indices.T.astype(jnp.int32)
    out_t = _make_gather(hist, batch)(idx_t, table)
    return jnp.transpose(out_t, (2, 0, 1))


# speedup vs baseline: 1.6102x; 1.2105x over previous
import functools

import jax
import jax.numpy as jnp
from jax import lax
from jax.experimental import pallas as pl
from jax.experimental.pallas import tpu as pltpu
from jax.experimental.pallas import tpu_sc as plsc

EMBED_DIM = 64
_NUM_CORES = 2
_NUM_SUBCORES = 16
_NW = _NUM_CORES * _NUM_SUBCORES
_CHUNK = 256
_LANES = 16


def _make_gather(hist, batch):
    stripe = batch // _NW
    per_h = stripe // _CHUNK
    nstep = hist * per_h
    assert nstep % 2 == 0 and nstep >= 4
    mesh = plsc.VectorSubcoreMesh(core_axis_name="c", subcore_axis_name="s")

    @functools.partial(
        pl.kernel,
        mesh=mesh,
        out_type=jax.ShapeDtypeStruct((hist, EMBED_DIM, batch), jnp.float32),
        scratch_types=[
            pltpu.VMEM((_CHUNK,), jnp.int32),
            pltpu.VMEM((_CHUNK,), jnp.int32),
            pltpu.VMEM((_CHUNK, 2 * EMBED_DIM), jnp.float32),
            pltpu.VMEM((_CHUNK, 2 * EMBED_DIM), jnp.float32),
            pltpu.VMEM((EMBED_DIM, _CHUNK), jnp.float32),
            pltpu.VMEM((EMBED_DIM, _CHUNK), jnp.float32),
            pltpu.SemaphoreType.DMA,
            pltpu.SemaphoreType.DMA,
            pltpu.SemaphoreType.DMA,
            pltpu.SemaphoreType.DMA,
            pltpu.SemaphoreType.DMA,
            pltpu.SemaphoreType.DMA,
        ],
        compiler_params=pltpu.CompilerParams(
            use_tc_tiling_on_sc=True, needs_layout_passes=False
        ),
    )
    def gather_kernel(idx_hbm, table_hbm, out_hbm,
                      idx0, idx1, rows0, rows1, tb0, tb1,
                      si0, si1, sg0, sg1, sw0, sw1):
        wid = lax.axis_index("s") * _NUM_CORES + lax.axis_index("c")
        b_base = wid * stripe
        idx_b, rows_b, tb_b = (idx0, idx1), (rows0, rows1), (tb0, tb1)
        si, sg, sw = (si0, si1), (sg0, sg1), (sw0, sw1)

        def chunk_pos(m):
            return m // per_h, b_base + (m % per_h) * _CHUNK

        def start_idx(m, b):
            h, b0 = chunk_pos(m)
            pltpu.make_async_copy(
                idx_hbm.at[h, pl.ds(b0, _CHUNK)], idx_b[b], si[b]
            ).start()

        def wait_idx(b):
            pltpu.make_async_copy(
                idx_hbm.at[0, pl.ds(0, _CHUNK)], idx_b[b], si[b]
            ).wait()

        def start_gather(b):
            pltpu.make_async_copy(table_hbm.at[idx_b[b]], rows_b[b], sg[b]).start()

        def wait_gather(b):
            pltpu.make_async_copy(table_hbm.at[idx_b[b]], rows_b[b], sg[b]).wait()

        def transpose(b):
            rows, tb = rows_b[b], tb_b[b]
            lane = lax.iota(jnp.int32, _LANES)
            rowidx = [lane + cg * _LANES for cg in range(EMBED_DIM // _LANES)]

            @plsc.parallel_loop(0, _CHUNK, unroll=8)
            def body(r):
                rvec = jnp.full((_LANES,), r, jnp.int32)
                for cg in range(EMBED_DIM // _LANES):
                    v = rows[r, pl.ds(cg * _LANES, _LANES)]
                    plsc.store_scatter(tb, [rowidx[cg], rvec], v)

        def start_wb(m, b):
            h, b0 = chunk_pos(m)
            pltpu.make_async_copy(
                tb_b[b], out_hbm.at[h].at[:, pl.ds(b0, _CHUNK)], sw[b]
            ).start()

        def wait_wb(b):
            pltpu.make_async_copy(
                tb_b[b], out_hbm.at[0].at[:, pl.ds(0, _CHUNK)], sw[b]
            ).wait()

        def steady(m, b, prefetch=True):
            o = 1 - b
            wait_idx(b)
            wait_wb(b)
            start_gather(b)
            wait_gather(o)
            if prefetch:
                start_idx(m + 1, o)
            transpose(o)
            start_wb(m - 1, o)

        start_idx(0, 0)
        wait_idx(0)
        start_gather(0)
        start_idx(1, 1)
        wait_idx(1)
        start_gather(1)
        wait_gather(0)
        start_idx(2, 0)
        transpose(0)
        start_wb(0, 0)

        def pair(p, carry):
            m = 2 * p
            steady(m, 0)
            steady(m + 1, 1)
            return carry

        lax.fori_loop(1, nstep // 2 - 1, pair, 0)

        steady(nstep - 2, 0)
        steady(nstep - 1, 1, prefetch=False)
        wait_gather(1)
        transpose(1)
        start_wb(nstep - 1, 1)
        wait_wb(0)
        wait_wb(1)

    return gather_kernel


def kernel(indices, table):
    batch, hist = indices.shape
    idx_t = indices.T.astype(jnp.int32)
    table128 = jnp.pad(table, ((0, 0), (0, 2 * EMBED_DIM - table.shape[1])))
    out_t = _make_gather(hist, batch)(idx_t, table128)
    return jnp.transpose(out_t, (2, 0, 1))


# speedup vs baseline: 1.6114x; 1.0008x over previous
import functools

import jax
import jax.numpy as jnp
from jax import lax
from jax.experimental import pallas as pl
from jax.experimental.pallas import tpu as pltpu
from jax.experimental.pallas import tpu_sc as plsc

EMBED_DIM = 64
_NUM_CORES = 2
_NUM_SUBCORES = 16
_NW = _NUM_CORES * _NUM_SUBCORES
_CHUNK = 256
_LANES = 16


def _make_gather(hist, batch):
    stripe = batch // _NW
    per_h = stripe // _CHUNK
    nstep = hist * per_h
    assert nstep % 2 == 0 and nstep >= 4
    mesh = plsc.VectorSubcoreMesh(core_axis_name="c", subcore_axis_name="s")

    @functools.partial(
        pl.kernel,
        mesh=mesh,
        out_type=jax.ShapeDtypeStruct((hist, EMBED_DIM, batch), jnp.float32),
        scratch_types=[
            pltpu.VMEM((_CHUNK,), jnp.int32),
            pltpu.VMEM((_CHUNK,), jnp.int32),
            pltpu.VMEM((_CHUNK, 2 * EMBED_DIM), jnp.float32),
            pltpu.VMEM((_CHUNK, 2 * EMBED_DIM), jnp.float32),
            pltpu.VMEM((EMBED_DIM, _CHUNK), jnp.float32),
            pltpu.VMEM((EMBED_DIM, _CHUNK), jnp.float32),
            pltpu.SemaphoreType.DMA,
            pltpu.SemaphoreType.DMA,
            pltpu.SemaphoreType.DMA,
            pltpu.SemaphoreType.DMA,
            pltpu.SemaphoreType.DMA,
            pltpu.SemaphoreType.DMA,
        ],
        compiler_params=pltpu.CompilerParams(
            use_tc_tiling_on_sc=True, needs_layout_passes=False
        ),
    )
    def gather_kernel(idx_hbm, table_hbm, out_hbm,
                      idx0, idx1, rows0, rows1, tb0, tb1,
                      si0, si1, sg0, sg1, sw0, sw1):
        wid = lax.axis_index("s") * _NUM_CORES + lax.axis_index("c")
        b_base = wid * stripe
        idx_b, rows_b, tb_b = (idx0, idx1), (rows0, rows1), (tb0, tb1)
        si, sg, sw = (si0, si1), (sg0, sg1), (sw0, sw1)

        def chunk_pos(m):
            return m // per_h, b_base + (m % per_h) * _CHUNK

        def start_idx(m, b):
            h, b0 = chunk_pos(m)
            pltpu.make_async_copy(
                idx_hbm.at[h, pl.ds(b0, _CHUNK)], idx_b[b], si[b]
            ).start()

        def wait_idx(b):
            pltpu.make_async_copy(
                idx_hbm.at[0, pl.ds(0, _CHUNK)], idx_b[b], si[b]
            ).wait()

        def start_gather(b):
            pltpu.make_async_copy(table_hbm.at[idx_b[b]], rows_b[b], sg[b]).start()

        def wait_gather(b):
            pltpu.make_async_copy(table_hbm.at[idx_b[b]], rows_b[b], sg[b]).wait()

        def transpose(b):
            rows, tb = rows_b[b], tb_b[b]
            lane = lax.iota(jnp.int32, _LANES)
            rowidx = [lane + cg * _LANES for cg in range(EMBED_DIM // _LANES)]

            @plsc.parallel_loop(0, _CHUNK, unroll=16)
            def body(r):
                rvec = jnp.full((_LANES,), r, jnp.int32)
                for cg in range(EMBED_DIM // _LANES):
                    v = rows[r, pl.ds(cg * _LANES, _LANES)]
                    plsc.store_scatter(tb, [rowidx[cg], rvec], v)

        def start_wb(m, b):
            h, b0 = chunk_pos(m)
            pltpu.make_async_copy(
                tb_b[b], out_hbm.at[h].at[:, pl.ds(b0, _CHUNK)], sw[b]
            ).start()

        def wait_wb(b):
            pltpu.make_async_copy(
                tb_b[b], out_hbm.at[0].at[:, pl.ds(0, _CHUNK)], sw[b]
            ).wait()

        def steady(m, b, prefetch=True):
            o = 1 - b
            wait_idx(b)
            wait_wb(b)
            start_gather(b)
            wait_gather(o)
            if prefetch:
                start_idx(m + 1, o)
            transpose(o)
            start_wb(m - 1, o)

        start_idx(0, 0)
        wait_idx(0)
        start_gather(0)
        start_idx(1, 1)
        wait_idx(1)
        start_gather(1)
        wait_gather(0)
        start_idx(2, 0)
        transpose(0)
        start_wb(0, 0)

        def pair(p, carry):
            m = 2 * p
            steady(m, 0)
            steady(m + 1, 1)
            return carry

        lax.fori_loop(1, nstep // 2 - 1, pair, 0)

        steady(nstep - 2, 0)
        steady(nstep - 1, 1, prefetch=False)
        wait_gather(1)
        transpose(1)
        start_wb(nstep - 1, 1)
        wait_wb(0)
        wait_wb(1)

    return gather_kernel


def kernel(indices, table):
    batch, hist = indices.shape
    idx_t = indices.T.astype(jnp.int32)
    table128 = jnp.pad(table, ((0, 0), (0, 2 * EMBED_DIM - table.shape[1])))
    out_t = _make_gather(hist, batch)(idx_t, table128)
    return jnp.transpose(out_t, (2, 0, 1))


# speedup vs baseline: 1.6711x; 1.0370x over previous
import functools

import jax
import jax.numpy as jnp
from jax import lax
from jax.experimental import pallas as pl
from jax.experimental.pallas import tpu as pltpu
from jax.experimental.pallas import tpu_sc as plsc

EMBED_DIM = 64
_NUM_CORES = 2
_NUM_SUBCORES = 16
_NW = _NUM_CORES * _NUM_SUBCORES
_CHUNK = 512


def _make_gather(batch):
    bpw = batch // _NW
    nstep = bpw // _CHUNK
    assert nstep % 2 == 0 and nstep >= 4
    mesh = plsc.VectorSubcoreMesh(core_axis_name="c", subcore_axis_name="s")

    @functools.partial(
        pl.kernel,
        mesh=mesh,
        out_type=jax.ShapeDtypeStruct((batch, EMBED_DIM), jnp.float32),
        scratch_types=[
            pltpu.VMEM((_CHUNK,), jnp.int32),
            pltpu.VMEM((_CHUNK,), jnp.int32),
            pltpu.VMEM((_CHUNK, EMBED_DIM), jnp.float32),
            pltpu.VMEM((_CHUNK, EMBED_DIM), jnp.float32),
            pltpu.SemaphoreType.DMA,
            pltpu.SemaphoreType.DMA,
            pltpu.SemaphoreType.DMA,
            pltpu.SemaphoreType.DMA,
            pltpu.SemaphoreType.DMA,
            pltpu.SemaphoreType.DMA,
        ],
        compiler_params=pltpu.CompilerParams(use_tc_tiling_on_sc=False),
    )
    def gather_kernel(idx_hbm, table_hbm, out_hbm,
                      idx0, idx1, rows0, rows1,
                      si0, si1, sg0, sg1, sw0, sw1):
        wid = lax.axis_index("s") * _NUM_CORES + lax.axis_index("c")
        base = wid * bpw
        idx_b, rows_b = (idx0, idx1), (rows0, rows1)
        si, sg, sw = (si0, si1), (sg0, sg1), (sw0, sw1)

        def start_idx(g, b):
            pltpu.make_async_copy(
                idx_hbm.at[pl.ds(base + g * _CHUNK, _CHUNK)], idx_b[b], si[b]
            ).start()

        def wait_idx(b):
            pltpu.make_async_copy(
                idx_hbm.at[pl.ds(base, _CHUNK)], idx_b[b], si[b]
            ).wait()

        def start_gather(b):
            pltpu.make_async_copy(table_hbm.at[idx_b[b]], rows_b[b], sg[b]).start()

        def wait_gather(b):
            pltpu.make_async_copy(table_hbm.at[idx_b[b]], rows_b[b], sg[b]).wait()

        def start_wb(g, b):
            pltpu.make_async_copy(
                rows_b[b], out_hbm.at[pl.ds(base + g * _CHUNK, _CHUNK)], sw[b]
            ).start()

        def wait_wb(b):
            pltpu.make_async_copy(
                rows_b[b], out_hbm.at[pl.ds(base, _CHUNK)], sw[b]
            ).wait()

        def steady(g, b):
            o = 1 - b
            wait_gather(o)
            start_wb(g - 1, o)
            start_idx(g + 1, o)
            wait_idx(b)
            wait_wb(b)
            start_gather(b)

        start_idx(0, 0)
        wait_idx(0)
        start_gather(0)
        start_idx(1, 1)
        wait_gather(0)
        start_wb(0, 0)
        start_idx(2, 0)
        wait_idx(1)
        start_gather(1)

        def pair(p, carry):
            g = 2 * p
            steady(g, 0)
            steady(g + 1, 1)
            return carry

        lax.fori_loop(1, nstep // 2 - 1, pair, 0)

        steady(nstep - 2, 0)
        wait_gather(0)
        start_wb(nstep - 2, 0)
        wait_idx(1)
        wait_wb(1)
        start_gather(1)
        wait_gather(1)
        start_wb(nstep - 1, 1)
        wait_wb(0)
        wait_wb(1)

    return gather_kernel


def kernel(indices, table):
    idx = indices.reshape(-1).astype(jnp.int32)
    out = _make_gather(idx.shape[0])(idx, table)
    return out.reshape(indices.shape + (EMBED_DIM,))
